# baseline (speedup 1.0000x reference)
import functools

import jax
import jax.numpy as jnp
from jax import lax
from jax.experimental import pallas as pl
from jax.experimental.pallas import tpu as pltpu
from jax.experimental.pallas import tpu_sc as plsc

_B = 16
_N = 256
_L = 256
_T = _B * _L
_TRI = _N * (_N + 1) // 2
_OUT1 = _N * _N
_HALF = _OUT1 // 2
_QUARTER = _OUT1 // 4
_NEG_INV_2SIG2 = -32768.0

_NB1 = 25
_N1 = _NB1 * 1024
_N2 = _TRI - _N1


def _elu(h):
    return jnp.where(h > 0, h, jnp.exp(h) - 1.0)


def _dot(a, b):
    return jnp.dot(a, b, preferred_element_type=jnp.float32)


def _prep_body(ptr_ref, x_ref, wca_ref, bca_ref, bins_ref,
               w1_ref, b1_ref, w2_ref, b2_ref, w3_ref, b3_ref,
               h3_ref, pada_ref, vals_ref, padc_ref):
    vals_ref[...] = _dot(x_ref[...], wca_ref[...]) + bca_ref[...]
    bins = bins_ref[...]

    def body(b, carry):
        off = ptr_ref[b]
        v = vals_ref[pl.ds(off, _L), :]
        vmin = jnp.min(v, axis=0, keepdims=True)
        vmax = jnp.max(v, axis=0, keepdims=True)
        vn = (v - vmin) * 2.0 / (vmax - vmin) - 1.0
        dc = vn[:, 0:1] - bins
        da = vn[:, 1:2] - bins
        hc = jnp.sum(jnp.exp(dc * dc * _NEG_INV_2SIG2), axis=0, keepdims=True)
        ha = jnp.sum(jnp.exp(da * da * _NEG_INV_2SIG2), axis=0, keepdims=True)
        padc_ref[pl.ds(b, 1), :] = hc
        pada_ref[pl.ds(b, 1), :] = ha
        return carry

    lax.fori_loop(0, _B, body, 0)

    h = _elu(_dot(padc_ref[...], w1_ref[...]) + b1_ref[...])
    h = _elu(_dot(h, w2_ref[...]) + b2_ref[...])
    h3_ref[...] = _elu(_dot(h, w3_ref[...]) + b3_ref[...])


_prep_call = pl.pallas_call(
    _prep_body,
    in_specs=[pl.BlockSpec(memory_space=pltpu.SMEM)] +
             [pl.BlockSpec(memory_space=pltpu.VMEM)] * 10,
    out_specs=[
        pl.BlockSpec(memory_space=pltpu.VMEM),
        pl.BlockSpec(memory_space=pltpu.VMEM),
    ],
    out_shape=[
        jax.ShapeDtypeStruct((_B, 1024), jnp.float32),
        jax.ShapeDtypeStruct((_B, _N), jnp.float32),
    ],
    scratch_shapes=[
        pltpu.VMEM((_T, 2), jnp.float32),
        pltpu.VMEM((_B, _N), jnp.float32),
    ],
)


def _atoms_body(pada_ref, a1_ref, c1_ref, a2_ref, c2_ref, a3_ref, c3_ref,
                a4_ref, c4_ref, x2_ref):
    g = _elu(_dot(pada_ref[...], a1_ref[...]) + c1_ref[...])
    g = _elu(_dot(g, a2_ref[...]) + c2_ref[...])
    g = _elu(_dot(g, a3_ref[...]) + c3_ref[...])
    x2_ref[...] = _dot(g, a4_ref[...]) + c4_ref[...]


_atoms_call = pl.pallas_call(
    _atoms_body,
    out_shape=jax.ShapeDtypeStruct((_B, 2816), jnp.float32),
)


def _wide_body(h_ref, w_ref, b_ref, out_ref):
    out_ref[...] = _dot(h_ref[...], w_ref[...]) + b_ref[...]


def _wide_call(h, w, b, tile_n, n_out=None, block_off=0):
    k, n = w.shape
    if n_out is None:
        n_out = n
    grid = pl.cdiv(n_out, tile_n)
    call = pl.pallas_call(
        _wide_body,
        grid=(grid,),
        in_specs=[
            pl.BlockSpec((_B, k), lambda t: (0, 0)),
            pl.BlockSpec((k, tile_n), lambda t: (0, t + block_off)),
            pl.BlockSpec((1, tile_n), lambda t: (0, t + block_off)),
        ],
        out_specs=pl.BlockSpec((_B, tile_n), lambda t: (0, t)),
        out_shape=jax.ShapeDtypeStruct((_B, n_out), jnp.float32),
        compiler_params=pltpu.CompilerParams(
            dimension_semantics=("arbitrary",)),
    )
    return call(h, w, b)


def _expand_body(diag_hbm, out_hbm, diag_v, out_v):
    cid = lax.axis_index("c")
    sid = lax.axis_index("s")
    b = sid
    half = cid
    pltpu.sync_copy(diag_hbm.at[b], diag_v)
    lanes = lax.iota(jnp.int32, 16)
    row0 = half * (_N // 2)
    cols = [cb * 16 + lanes for cb in range(16)]
    flow = [c * 255 - jnp.right_shift(c * (c - 1), 1) for c in cols]

    @plsc.parallel_loop(0, _N // 2, step=1, unroll=4)
    def body(i):
        r = row0 + i
        fr = r * 255 - jnp.right_shift(r * (r - 1), 1)
        base = i * _N
        for cb in range(16):
            k = jnp.where(cols[cb] < r, flow[cb] + r, cols[cb] + fr)
            out_v[pl.ds(base + cb * 16, 16)] = plsc.load_gather(diag_v, [k])

    pltpu.sync_copy(out_v, out_hbm.at[b, pl.ds(half * _HALF, _HALF)])


@functools.cache
def _expand_call():
    return functools.partial(
        pl.kernel,
        out_type=jax.ShapeDtypeStruct((_B, _OUT1), jnp.float32),
        mesh=plsc.VectorSubcoreMesh(core_axis_name="c", subcore_axis_name="s",
                                    num_cores=2, num_subcores=16),
        scratch_types=[
            pltpu.VMEM((_TRI,), jnp.float32),
            pltpu.VMEM((_HALF,), jnp.float32),
        ],
        compiler_params=pltpu.CompilerParams(needs_layout_passes=False),
    )(_expand_body)


def kernel(x, ptr, Wc, bc, Wa, ba, cmap_params, atoms_params):
    w1, b1, w2, b2, w3, b3, w4, b4 = cmap_params
    a1, c1, a2, c2, a3, c3, a4, c4 = atoms_params

    wca = jnp.concatenate([Wc, Wa], axis=1)
    bca = jnp.concatenate([bc, ba]).reshape(1, 2)
    bins = (jnp.linspace(-1.0, 1.0, _N + 1)[1:] + 0.1 * 0.5).reshape(1, _N)
    b4r = b4.reshape(1, -1)

    h3, pada = _prep_call(
        ptr, x, wca, bca, bins,
        w1, b1.reshape(1, -1), w2, b2.reshape(1, -1), w3, b3.reshape(1, -1))

    diag = _wide_call(h3, w4, b4r, 1024)

    x2 = _atoms_call(pada, a1, c1.reshape(1, -1), a2, c2.reshape(1, -1),
                     a3, c3.reshape(1, -1), a4, c4.reshape(1, -1))

    out1 = _expand_call()(diag)
    return out1, x2


# speedup vs baseline: 1.0913x; 1.0913x over previous
import functools

import jax
import jax.numpy as jnp
from jax import lax
from jax.experimental import pallas as pl
from jax.experimental.pallas import tpu as pltpu
from jax.experimental.pallas import tpu_sc as plsc

_B = 16
_N = 256
_L = 256
_T = _B * _L
_TRI = _N * (_N + 1) // 2
_OUT1 = _N * _N
_HALF = _OUT1 // 2
_QUARTER = _OUT1 // 4
_NEG_INV_2SIG2 = -32768.0

_NB1 = 25
_N1 = _NB1 * 1024
_N2 = _TRI - _N1


def _elu(h):
    return jnp.where(h > 0, h, jnp.exp(h) - 1.0)


def _dot(a, b):
    return jnp.dot(a, b, preferred_element_type=jnp.float32)


def _prep_body(ptr_ref, x_ref, wca_ref, bca_ref, bins_ref,
               w1_ref, b1_ref, w2_ref, b2_ref, w3_ref, b3_ref,
               h3_ref, pada_ref, vals_ref, padc_ref):
    vals_ref[...] = _dot(x_ref[...], wca_ref[...]) + bca_ref[...]
    bins = bins_ref[...]

    def body(b, carry):
        off = ptr_ref[b]
        v = vals_ref[pl.ds(off, _L), :]
        vmin = jnp.min(v, axis=0, keepdims=True)
        vmax = jnp.max(v, axis=0, keepdims=True)
        vn = (v - vmin) * 2.0 / (vmax - vmin) - 1.0
        dc = vn[:, 0:1] - bins
        da = vn[:, 1:2] - bins
        hc = jnp.sum(jnp.exp(dc * dc * _NEG_INV_2SIG2), axis=0, keepdims=True)
        ha = jnp.sum(jnp.exp(da * da * _NEG_INV_2SIG2), axis=0, keepdims=True)
        padc_ref[pl.ds(b, 1), :] = hc
        pada_ref[pl.ds(b, 1), :] = ha
        return carry

    lax.fori_loop(0, _B, body, 0)

    h = _elu(_dot(padc_ref[...], w1_ref[...]) + b1_ref[...])
    h = _elu(_dot(h, w2_ref[...]) + b2_ref[...])
    h3_ref[...] = _elu(_dot(h, w3_ref[...]) + b3_ref[...])


_prep_call = pl.pallas_call(
    _prep_body,
    in_specs=[pl.BlockSpec(memory_space=pltpu.SMEM)] +
             [pl.BlockSpec(memory_space=pltpu.VMEM)] * 10,
    out_specs=[
        pl.BlockSpec(memory_space=pltpu.VMEM),
        pl.BlockSpec(memory_space=pltpu.VMEM),
    ],
    out_shape=[
        jax.ShapeDtypeStruct((_B, 1024), jnp.float32),
        jax.ShapeDtypeStruct((_B, _N), jnp.float32),
    ],
    scratch_shapes=[
        pltpu.VMEM((_T, 2), jnp.float32),
        pltpu.VMEM((_B, _N), jnp.float32),
    ],
)


def _atoms_body(pada_ref, a1_ref, c1_ref, a2_ref, c2_ref, a3_ref, c3_ref,
                a4_ref, c4_ref, x2_ref):
    g = _elu(_dot(pada_ref[...], a1_ref[...]) + c1_ref[...])
    g = _elu(_dot(g, a2_ref[...]) + c2_ref[...])
    g = _elu(_dot(g, a3_ref[...]) + c3_ref[...])
    x2_ref[...] = _dot(g, a4_ref[...]) + c4_ref[...]


_atoms_call = pl.pallas_call(
    _atoms_body,
    out_shape=jax.ShapeDtypeStruct((_B, 2816), jnp.float32),
)


def _wide_body(h_ref, w_ref, b_ref, out_ref):
    out_ref[...] = _dot(h_ref[...], w_ref[...]) + b_ref[...]


def _wide_call(h, w, b, tile_n, n_out=None, block_off=0):
    k, n = w.shape
    if n_out is None:
        n_out = n
    grid = pl.cdiv(n_out, tile_n)
    call = pl.pallas_call(
        _wide_body,
        grid=(grid,),
        in_specs=[
            pl.BlockSpec((_B, k), lambda t: (0, 0)),
            pl.BlockSpec((k, tile_n), lambda t: (0, t + block_off)),
            pl.BlockSpec((1, tile_n), lambda t: (0, t + block_off)),
        ],
        out_specs=pl.BlockSpec((_B, tile_n), lambda t: (0, t)),
        out_shape=jax.ShapeDtypeStruct((_B, n_out), jnp.float32),
        compiler_params=pltpu.CompilerParams(
            dimension_semantics=("arbitrary",)),
    )
    return call(h, w, b)


def _expand_body(diag_hbm, out_hbm, diag_v, out_v):
    cid = lax.axis_index("c")
    sid = lax.axis_index("s")
    b = sid
    half = cid
    pltpu.sync_copy(diag_hbm.at[b], diag_v)
    lanes = lax.iota(jnp.int32, 16)
    row0 = half * (_N // 2)
    cols = [cb * 16 + lanes for cb in range(16)]
    flow = [c * 255 - jnp.right_shift(c * (c - 1), 1) for c in cols]

    @plsc.parallel_loop(0, _N // 2, step=1, unroll=2)
    def body(i):
        r = row0 + i
        fr = r * 255 - jnp.right_shift(r * (r - 1), 1)
        base = i * _N
        for cb in range(16):
            k = jnp.where(cols[cb] < r, flow[cb] + r, cols[cb] + fr)
            out_v[pl.ds(base + cb * 16, 16)] = plsc.load_gather(diag_v, [k])

    pltpu.sync_copy(out_v, out_hbm.at[b, pl.ds(half * _HALF, _HALF)])


@functools.cache
def _expand_call():
    return functools.partial(
        pl.kernel,
        out_type=jax.ShapeDtypeStruct((_B, _OUT1), jnp.float32),
        mesh=plsc.VectorSubcoreMesh(core_axis_name="c", subcore_axis_name="s",
                                    num_cores=2, num_subcores=16),
        scratch_types=[
            pltpu.VMEM((_TRI,), jnp.float32),
            pltpu.VMEM((_HALF,), jnp.float32),
        ],
        compiler_params=pltpu.CompilerParams(needs_layout_passes=False),
    )(_expand_body)


def kernel(x, ptr, Wc, bc, Wa, ba, cmap_params, atoms_params):
    w1, b1, w2, b2, w3, b3, w4, b4 = cmap_params
    a1, c1, a2, c2, a3, c3, a4, c4 = atoms_params

    wca = jnp.concatenate([Wc, Wa], axis=1)
    bca = jnp.concatenate([bc, ba]).reshape(1, 2)
    bins = (jnp.linspace(-1.0, 1.0, _N + 1)[1:] + 0.1 * 0.5).reshape(1, _N)
    b4r = b4.reshape(1, -1)

    h3, pada = _prep_call(
        ptr, x, wca, bca, bins,
        w1, b1.reshape(1, -1), w2, b2.reshape(1, -1), w3, b3.reshape(1, -1))

    diag = _wide_call(h3, w4, b4r, 2048)

    x2 = _atoms_call(pada, a1, c1.reshape(1, -1), a2, c2.reshape(1, -1),
                     a3, c3.reshape(1, -1), a4, c4.reshape(1, -1))

    out1 = _expand_call()(diag)
    return out1, x2
